# Initial kernel scaffold; baseline (speedup 1.0000x reference)
import jax
import jax.numpy as jnp
from jax.experimental import pallas as pl


def kernel(x, w_conv2d, bn1_scale, bn1_shift, w_conv3d_1, bn2_scale, bn2_shift, w_conv3d_2, bn3_scale, bn3_shift, w_fc, b_fc):
    raise NotImplementedError("write your pallas kernel here")


# speedup vs baseline: 24.4924x; 24.4924x over previous
import functools

import jax
import jax.numpy as jnp
from jax.experimental import pallas as pl
from jax.experimental.pallas import tpu as pltpu


def _rup(x, m):
    return (x + m - 1) // m * m


def _s1_body(a_ref, b_ref, t_ref, o_ref):
    acc = jnp.dot(a_ref[...], b_ref[...], preferred_element_type=jnp.float32)
    o_ref[...] = jnp.maximum(acc + t_ref[...], 0.0).astype(o_ref.dtype)


def _conv2d_packed(patch, w1p, shift16, n_tiles):
    M, K = patch.shape
    N = w1p.shape[1]
    tm = M // n_tiles
    return pl.pallas_call(
        _s1_body,
        out_shape=jax.ShapeDtypeStruct((M, N), jnp.bfloat16),
        grid=(n_tiles,),
        in_specs=[
            pl.BlockSpec((tm, K), lambda i: (i, 0)),
            pl.BlockSpec((K, N), lambda i: (0, 0)),
            pl.BlockSpec((1, N), lambda i: (0, 0)),
        ],
        out_specs=pl.BlockSpec((tm, N), lambda i: (i, 0)),
        compiler_params=pltpu.CompilerParams(
            dimension_semantics=("parallel",)),
    )(patch, w1p, shift16)


def _conv3d_body(z_ref, w_ref, t_ref, o_ref, a_ref, prod_ref, *,
                 tap_offs, n_in, n_out, fs, cz, kt, cout, pool):
    n_g = 4
    fpg = [n_in // n_g + (1 if g < n_in % n_g else 0) for g in range(n_g)]
    base = 0
    for g in range(n_g):
        gr = fpg[g] * fs
        for k, off in enumerate(tap_offs):
            a_ref[pl.ds(base, gr), k * cz:(k + 1) * cz] = (
                z_ref[0, pl.ds(base + off, gr), :])
        prod_ref[pl.ds(base, gr), :] = jnp.dot(
            a_ref[pl.ds(base, gr), :], w_ref[...],
            preferred_element_type=jnp.float32)
        base += gr
    m = n_out * fs
    acc = prod_ref[pl.ds(0, m), pl.ds(0, cout)]
    for dt in range(1, kt):
        acc = acc + prod_ref[pl.ds(dt * fs, m), pl.ds(dt * cout, cout)]
    if pool:
        cur = acc[0:fs, :]
        for ti in range(1, n_out):
            cur = jnp.maximum(cur, acc[ti * fs:(ti + 1) * fs, :])
        o_ref[0] = jnp.maximum(cur + t_ref[...], 0.0)
    else:
        o_ref[0] = jnp.maximum(acc + t_ref[...], 0.0).astype(o_ref.dtype)


def _conv3d_parity(z, w, shift, *, gw, n_in, n_out, fs, cz, kt, cout, pool):
    B, Zrows, _ = z.shape
    tap_offs = tuple(kh * gw + kw for kh in range(3) for kw in range(3))
    body = functools.partial(
        _conv3d_body, tap_offs=tap_offs, n_in=n_in, n_out=n_out, fs=fs,
        cz=cz, kt=kt, cout=cout, pool=pool)
    if pool:
        out_shape = jax.ShapeDtypeStruct((B, fs, cout), jnp.float32)
        out_spec = pl.BlockSpec((1, fs, cout), lambda b: (b, 0, 0))
    else:
        out_shape = jax.ShapeDtypeStruct((B, n_out * fs, cout), jnp.bfloat16)
        out_spec = pl.BlockSpec((1, n_out * fs, cout), lambda b: (b, 0, 0))
    return pl.pallas_call(
        body,
        out_shape=out_shape,
        grid=(B,),
        in_specs=[
            pl.BlockSpec((1, Zrows, cz), lambda b: (b, 0, 0)),
            pl.BlockSpec(w.shape, lambda b: (0, 0)),
            pl.BlockSpec((1, cout), lambda b: (0, 0)),
        ],
        out_specs=out_spec,
        scratch_shapes=[
            pltpu.VMEM((n_in * fs, 9 * cz), jnp.bfloat16),
            pltpu.VMEM((n_in * fs, kt * cout), jnp.float32),
        ],
        compiler_params=pltpu.CompilerParams(
            dimension_semantics=("parallel",),
            vmem_limit_bytes=60 << 20,
        ),
    )(z, w, shift.reshape(1, cout))


def _space_to_depth(y, pad_t, pad_back):
    B, T, H, W, C = y.shape
    yp = jnp.pad(y, ((0, 0), (pad_t, pad_t), (2, pad_back), (2, pad_back),
                     (0, 0)))
    Hp = (H + 2 + pad_back) // 2
    yp = yp.reshape(B, T + 2 * pad_t, Hp, 2, Hp, 2, C)
    return jnp.transpose(yp, (0, 1, 2, 4, 3, 5, 6))


def _parity_rows(z6, g, fs, zrows):
    B, F = z6.shape[0], z6.shape[1]
    C4 = z6.shape[4] * z6.shape[5] * z6.shape[6]
    z = z6[:, :, :g, :g].reshape(B, F, g * g, C4)
    z = jnp.pad(z, ((0, 0), (0, 0), (0, fs - g * g), (0, 0)))
    z = z.reshape(B, F * fs, C4)
    return jnp.pad(z, ((0, 0), (0, zrows - F * fs), (0, 0)))


def _parity_weight(w, scale):
    Cout, Cin, kt = w.shape[0], w.shape[1], w.shape[2]
    wp = jnp.pad(w, ((0, 0), (0, 0), (0, 0), (1, 0), (1, 0)))
    wp = wp.reshape(Cout, Cin, kt, 3, 2, 3, 2)
    wp = jnp.transpose(wp, (3, 5, 4, 6, 1, 2, 0)) * scale
    return wp.reshape(9 * 4 * Cin, kt * Cout).astype(jnp.bfloat16)


def kernel(x, w_conv2d, bn1_scale, bn1_shift, w_conv3d_1, bn2_scale, bn2_shift,
           w_conv3d_2, bn3_scale, bn3_shift, w_fc, b_fc):
    B, C, T, H, W = x.shape
    xb = x[:, 0].astype(jnp.bfloat16)

    Ho1 = (H - 1) // 2 + 1
    Wo1 = (W - 1) // 2 + 1
    xp = jnp.pad(xb.reshape(B * T, H, W), ((0, 0), (1, 1), (1, 1)))
    taps = [xp[:, dh:dh + 2 * (Ho1 - 1) + 1:2, dw:dw + 2 * (Wo1 - 1) + 1:2]
            for dh in range(3) for dw in range(3)]
    patch = jnp.stack(taps, axis=-1).reshape(B * T * Ho1 * Wo1, 9)
    P = 16
    m_pos = patch.shape[0]
    n_tiles = 8
    rows = -(-m_pos // P)
    tm = _rup(-(-rows // n_tiles), 16)
    m_pad = tm * n_tiles
    patch = jnp.pad(patch, ((0, m_pad * P - m_pos), (0, 0))).reshape(m_pad, P * 9)

    w1 = jnp.transpose(w_conv2d, (2, 3, 1, 0)).reshape(9, 16) * bn1_scale[None, :]
    w1p = jnp.einsum("pq,kc->pkqc", jnp.eye(P, dtype=jnp.float32), w1)
    w1p = w1p.reshape(P * 9, P * 16).astype(jnp.bfloat16)
    y = _conv2d_packed(patch, w1p, jnp.tile(bn1_shift, P).reshape(1, P * 16),
                       n_tiles)
    y = y.reshape(m_pad * P, 16)[:m_pos].reshape(B, T, Ho1, Wo1, 16)

    g2 = Ho1 // 2 + 1
    fs2 = _rup(g2 * g2, 16)
    To2, F2 = T - 2, T + 2
    zrows2 = _rup(F2 * fs2 + 2 * g2 + 2 + 8, 16)
    z2 = _parity_rows(_space_to_depth(y, 1, 2), g2, fs2, zrows2)
    w2 = _parity_weight(w_conv3d_1, bn2_scale)
    y2 = _conv3d_parity(z2, w2, bn2_shift, gw=g2, n_in=F2, n_out=To2,
                        fs=fs2, cz=64, kt=5, cout=32, pool=False)
    Ho2 = (Ho1 - 3) // 2 + 1
    y2 = y2.reshape(B, To2, fs2, 32)[:, :, :g2 * g2]
    y2 = y2.reshape(B, To2, g2, g2, 32)[:, :, :Ho2, :Ho2]

    g3 = (Ho2 + 1) // 2 + 1
    fs3 = _rup(g3 * g3, 16)
    To3, F3 = To2 - 2, To2 + 2
    zrows3 = _rup(F3 * fs3 + 2 * g3 + 2 + 8, 16)
    z3 = _parity_rows(_space_to_depth(y2, 1, 3), g3, fs3, zrows3)
    w3 = _parity_weight(w_conv3d_2, bn3_scale)
    pooled = _conv3d_parity(z3, w3, bn3_shift, gw=g3, n_in=F3, n_out=To3,
                            fs=fs3, cz=128, kt=5, cout=64, pool=True)

    Ho3 = 8
    feat = pooled[:, :g3 * g3].reshape(B, g3, g3, 64)[:, :Ho3, :Ho3]
    feat = jnp.transpose(feat, (0, 3, 1, 2)).reshape(B, 64 * Ho3 * Ho3)
    return feat @ w_fc.T + b_fc
